# baseline (speedup 1.0000x reference)
import functools

import jax
import jax.numpy as jnp
from jax import lax
from jax.experimental import pallas as pl
from jax.experimental.pallas import tpu as pltpu
from jax.experimental.pallas import tpu_sc as plsc

_F = 16


@functools.lru_cache(maxsize=None)
def _build(batch):
    info = plsc.get_sparse_core_info()
    nc, ns = info.num_cores, info.num_subcores
    nw = nc * ns
    assert batch % (8 * nw) == 0
    bpw = batch // nw
    ngrp = bpw // 16

    mesh = plsc.VectorSubcoreMesh(core_axis_name="c", subcore_axis_name="s")

    @functools.partial(
        pl.kernel,
        mesh=mesh,
        out_type=jax.ShapeDtypeStruct((batch,), jnp.float32),
        compiler_params=pltpu.CompilerParams(
            needs_layout_passes=False, use_tc_tiling_on_sc=False),
        scratch_types=[
            pltpu.VMEM((bpw,), jnp.int32),
            pltpu.VMEM((bpw,), jnp.int32),
            pltpu.VMEM((_F, bpw), jnp.float32),
            pltpu.VMEM((_F, bpw), jnp.float32),
            pltpu.VMEM((bpw,), jnp.float32),
            pltpu.VMEM((bpw,), jnp.float32),
            pltpu.VMEM((bpw,), jnp.float32),
            pltpu.VMEM((bpw,), jnp.float32),
            pltpu.SemaphoreType.DMA,
            pltpu.SemaphoreType.DMA,
            pltpu.SemaphoreType.DMA,
        ],
    )
    def cdcf_kernel(user_hbm, item_hbm, avg_hbm, eu_hbm, ei_hbm,
                    bu_hbm, bi_hbm, out_hbm,
                    uidx_v, iidx_v, ue_v, ie_v, bu_v, bi_v, avg_v, out_v,
                    sem_ue, sem_ie, sem_b):
        wid = lax.axis_index("s") * nc + lax.axis_index("c")
        base = wid * bpw

        pltpu.sync_copy(user_hbm.at[pl.ds(base, bpw)], uidx_v)
        pltpu.sync_copy(item_hbm.at[pl.ds(base, bpw)], iidx_v)

        cbu = pltpu.async_copy(bu_hbm.at[uidx_v], bu_v, sem_b)
        cbi = pltpu.async_copy(bi_hbm.at[iidx_v], bi_v, sem_b)

        copies = []
        for f in range(_F):
            copies.append(pltpu.async_copy(
                eu_hbm.at[f].at[uidx_v], ue_v.at[f], sem_ue))
            copies.append(pltpu.async_copy(
                ei_hbm.at[f].at[iidx_v], ie_v.at[f], sem_ie))

        pltpu.sync_copy(avg_hbm.at[pl.ds(base, bpw)], avg_v)

        cbu.wait()
        cbi.wait()
        for c in copies:
            c.wait()

        def red(g, carry):
            s = pl.ds(g * 16, 16)
            acc = avg_v[s] + bu_v[s] + bi_v[s]
            for f in range(_F):
                acc = acc + ue_v[f, s] * ie_v[f, s]
            out_v[s] = acc
            return carry

        lax.fori_loop(0, ngrp, red, 0)

        pltpu.sync_copy(out_v, out_hbm.at[pl.ds(base, bpw)])

    return cdcf_kernel


def kernel(user, item, average, embed_user, embed_item, user_bias, item_bias):
    user = user.astype(jnp.int32)
    item = item.astype(jnp.int32)
    fn = _build(user.shape[0])
    return fn(user, item, average, embed_user.T, embed_item.T,
              user_bias, item_bias)


# speedup vs baseline: 17.8291x; 17.8291x over previous
import functools

import jax
import jax.numpy as jnp
from jax import lax
from jax.experimental import pallas as pl
from jax.experimental.pallas import tpu as pltpu
from jax.experimental.pallas import tpu_sc as plsc

_F = 16
_K = 16


@functools.lru_cache(maxsize=None)
def _build(batch):
    info = plsc.get_sparse_core_info()
    nc, ns = info.num_cores, info.num_subcores
    nw = nc * ns
    assert batch % (8 * nw) == 0
    bpw = batch // nw
    nchunk = bpw // _K

    mesh = plsc.VectorSubcoreMesh(core_axis_name="c", subcore_axis_name="s")

    @functools.partial(
        pl.kernel,
        mesh=mesh,
        out_type=jax.ShapeDtypeStruct((batch,), jnp.float32),
        compiler_params=pltpu.CompilerParams(needs_layout_passes=False),
        scratch_types=[
            pltpu.VMEM((bpw,), jnp.int32),
            pltpu.VMEM((bpw,), jnp.int32),
            pltpu.VMEM((_K, _F, 128), jnp.float32),
            pltpu.VMEM((_K, _F, 128), jnp.float32),
            pltpu.VMEM((bpw,), jnp.float32),
            pltpu.VMEM((bpw,), jnp.float32),
            pltpu.VMEM((bpw,), jnp.float32),
            pltpu.VMEM((bpw,), jnp.float32),
            pltpu.SemaphoreType.DMA,
            pltpu.SemaphoreType.DMA,
            pltpu.SemaphoreType.DMA,
        ],
    )
    def cdcf_kernel(user_hbm, item_hbm, avg_hbm, eu_hbm, ei_hbm,
                    bu_hbm, bi_hbm, out_hbm,
                    uidx_v, iidx_v, uslab_v, islab_v,
                    bu_v, bi_v, avg_v, out_v,
                    sem_ue, sem_ie, sem_b):
        wid = lax.axis_index("s") * nc + lax.axis_index("c")
        base = wid * bpw

        pltpu.sync_copy(user_hbm.at[pl.ds(base, bpw)], uidx_v)
        pltpu.sync_copy(item_hbm.at[pl.ds(base, bpw)], iidx_v)

        cbu = pltpu.async_copy(bu_hbm.at[uidx_v], bu_v, sem_b)
        cbi = pltpu.async_copy(bi_hbm.at[iidx_v], bi_v, sem_b)

        pltpu.sync_copy(avg_hbm.at[pl.ds(base, bpw)], avg_v)
        cbu.wait()
        cbi.wait()

        lane = lax.iota(jnp.int32, 16)

        def chunk(c, carry):
            s = pl.ds(c * _K, _K)
            u16 = uidx_v[s]
            i16 = iidx_v[s]
            copies = []
            for k in range(_K):
                ucol = pl.multiple_of((u16[k] >> 7) * 128, 128)
                icol = pl.multiple_of((i16[k] >> 7) * 128, 128)
                copies.append(pltpu.async_copy(
                    eu_hbm.at[:, pl.ds(ucol, 128)], uslab_v.at[k], sem_ue))
                copies.append(pltpu.async_copy(
                    ei_hbm.at[:, pl.ds(icol, 128)], islab_v.at[k], sem_ie))
            for cp in copies:
                cp.wait()
            acc = avg_v[s] + bu_v[s] + bi_v[s]
            for k in range(_K):
                kk = jnp.full((16,), k, jnp.int32)
                uc = jnp.broadcast_to(u16[k] & 127, (16,))
                ic = jnp.broadcast_to(i16[k] & 127, (16,))
                uf = plsc.load_gather(uslab_v, [kk, lane, uc])
                vf = plsc.load_gather(islab_v, [kk, lane, ic])
                dot = jnp.sum(uf * vf, axis=0)
                acc = jnp.where(lane == k, acc + dot, acc)
            out_v[s] = acc
            return carry

        lax.fori_loop(0, nchunk, chunk, 0)

        pltpu.sync_copy(out_v, out_hbm.at[pl.ds(base, bpw)])

    return cdcf_kernel


def kernel(user, item, average, embed_user, embed_item, user_bias, item_bias):
    user = user.astype(jnp.int32)
    item = item.astype(jnp.int32)
    fn = _build(user.shape[0])
    return fn(user, item, average, embed_user.T, embed_item.T,
              user_bias, item_bias)


# speedup vs baseline: 18.1059x; 1.0155x over previous
import functools

import jax
import jax.numpy as jnp
from jax import lax
from jax.experimental import pallas as pl
from jax.experimental.pallas import tpu as pltpu
from jax.experimental.pallas import tpu_sc as plsc

_F = 16


@functools.lru_cache(maxsize=None)
def _build(batch):
    info = plsc.get_sparse_core_info()
    nc, ns = info.num_cores, info.num_subcores
    nw = nc * ns
    assert batch % (8 * nw) == 0
    bpw = batch // nw
    ngrp = bpw // 16

    mesh = plsc.VectorSubcoreMesh(core_axis_name="c", subcore_axis_name="s")

    @functools.partial(
        pl.kernel,
        mesh=mesh,
        out_type=jax.ShapeDtypeStruct((batch,), jnp.float32),
        compiler_params=pltpu.CompilerParams(needs_layout_passes=False),
        scratch_types=[
            pltpu.VMEM((bpw,), jnp.int32),
            pltpu.VMEM((bpw,), jnp.int32),
            pltpu.VMEM((8, _F, 128), jnp.float32),
            pltpu.VMEM((8, _F, 128), jnp.float32),
            pltpu.VMEM((8, _F, 128), jnp.float32),
            pltpu.VMEM((8, _F, 128), jnp.float32),
            pltpu.VMEM((bpw,), jnp.float32),
            pltpu.VMEM((bpw,), jnp.float32),
            pltpu.VMEM((bpw,), jnp.float32),
            pltpu.VMEM((bpw,), jnp.float32),
            pltpu.SemaphoreType.DMA,
            pltpu.SemaphoreType.DMA,
            pltpu.SemaphoreType.DMA,
        ],
    )
    def cdcf_kernel(user_hbm, item_hbm, avg_hbm, eu_hbm, ei_hbm,
                    bu_hbm, bi_hbm, out_hbm,
                    uidx_v, iidx_v, ua_v, ub_v, ia_v, ib_v,
                    bu_v, bi_v, avg_v, out_v,
                    sem_ue, sem_ie, sem_b):
        wid = lax.axis_index("s") * nc + lax.axis_index("c")
        base = wid * bpw

        pltpu.sync_copy(user_hbm.at[pl.ds(base, bpw)], uidx_v)
        pltpu.sync_copy(item_hbm.at[pl.ds(base, bpw)], iidx_v)

        cbu = pltpu.async_copy(bu_hbm.at[uidx_v], bu_v, sem_b)
        cbi = pltpu.async_copy(bi_hbm.at[iidx_v], bi_v, sem_b)

        pltpu.sync_copy(avg_hbm.at[pl.ds(base, bpw)], avg_v)
        cbu.wait()
        cbi.wait()

        lane = lax.iota(jnp.int32, 16)

        def fire(u16, i16, lo, udst, idst):
            cps = []
            for k in range(lo, lo + 8):
                ucol = pl.multiple_of((u16[k] >> 7) * 128, 128)
                icol = pl.multiple_of((i16[k] >> 7) * 128, 128)
                cps.append(pltpu.async_copy(
                    eu_hbm.at[:, pl.ds(ucol, 128)], udst.at[k - lo], sem_ue))
                cps.append(pltpu.async_copy(
                    ei_hbm.at[:, pl.ds(icol, 128)], idst.at[k - lo], sem_ie))
            return cps

        def accumulate(u16, i16, lo, usrc, isrc, acc):
            for k in range(lo, lo + 8):
                kk = jnp.full((16,), k - lo, jnp.int32)
                uc = jnp.broadcast_to(u16[k] & 127, (16,))
                ic = jnp.broadcast_to(i16[k] & 127, (16,))
                uf = plsc.load_gather(usrc, [kk, lane, uc])
                vf = plsc.load_gather(isrc, [kk, lane, ic])
                dot = jnp.sum(uf * vf, axis=0)
                acc = jnp.where(lane == k, acc + dot, acc)
            return acc

        def grp(g, carry):
            s = pl.ds(g * 16, 16)
            u16 = uidx_v[s]
            i16 = iidx_v[s]
            cps_a = fire(u16, i16, 0, ua_v, ia_v)
            cps_b = fire(u16, i16, 8, ub_v, ib_v)
            acc = avg_v[s] + bu_v[s] + bi_v[s]
            for cp in cps_a:
                cp.wait()
            acc = accumulate(u16, i16, 0, ua_v, ia_v, acc)
            for cp in cps_b:
                cp.wait()
            acc = accumulate(u16, i16, 8, ub_v, ib_v, acc)
            out_v[s] = acc
            return carry

        lax.fori_loop(0, ngrp, grp, 0)

        pltpu.sync_copy(out_v, out_hbm.at[pl.ds(base, bpw)])

    return cdcf_kernel


def kernel(user, item, average, embed_user, embed_item, user_bias, item_bias):
    user = user.astype(jnp.int32)
    item = item.astype(jnp.int32)
    fn = _build(user.shape[0])
    return fn(user, item, average, embed_user.T, embed_item.T,
              user_bias, item_bias)


# speedup vs baseline: 18.1160x; 1.0006x over previous
import functools

import jax
import jax.numpy as jnp
from jax import lax
from jax.experimental import pallas as pl
from jax.experimental.pallas import tpu as pltpu
from jax.experimental.pallas import tpu_sc as plsc

_F = 16


@functools.lru_cache(maxsize=None)
def _build(batch):
    info = plsc.get_sparse_core_info()
    nc, ns = info.num_cores, info.num_subcores
    nw = nc * ns
    assert batch % (8 * nw) == 0
    bpw = batch // nw
    ngrp = bpw // 16

    mesh = plsc.VectorSubcoreMesh(core_axis_name="c", subcore_axis_name="s")

    @functools.partial(
        pl.kernel,
        mesh=mesh,
        out_type=jax.ShapeDtypeStruct((batch,), jnp.float32),
        compiler_params=pltpu.CompilerParams(needs_layout_passes=False),
        scratch_types=[
            pltpu.VMEM((bpw,), jnp.int32),
            pltpu.VMEM((bpw,), jnp.int32),
            pltpu.VMEM((8, _F, 128), jnp.float32),
            pltpu.VMEM((8, _F, 128), jnp.float32),
            pltpu.VMEM((8, _F, 128), jnp.float32),
            pltpu.VMEM((8, _F, 128), jnp.float32),
            pltpu.VMEM((bpw,), jnp.float32),
            pltpu.VMEM((bpw,), jnp.float32),
            pltpu.VMEM((bpw,), jnp.float32),
            pltpu.VMEM((bpw,), jnp.float32),
            pltpu.SemaphoreType.DMA,
            pltpu.SemaphoreType.DMA,
            pltpu.SemaphoreType.DMA,
        ],
    )
    def cdcf_kernel(user_hbm, item_hbm, avg_hbm, eu_hbm, ei_hbm,
                    bu_hbm, bi_hbm, out_hbm,
                    uidx_v, iidx_v, ua_v, ub_v, ia_v, ib_v,
                    bu_v, bi_v, avg_v, out_v,
                    sem_ue, sem_ie, sem_b):
        wid = lax.axis_index("s") * nc + lax.axis_index("c")
        base = wid * bpw

        pltpu.sync_copy(user_hbm.at[pl.ds(base, bpw)], uidx_v)
        pltpu.sync_copy(item_hbm.at[pl.ds(base, bpw)], iidx_v)

        cbu = pltpu.async_copy(bu_hbm.at[uidx_v], bu_v, sem_b)
        cbi = pltpu.async_copy(bi_hbm.at[iidx_v], bi_v, sem_b)

        pltpu.sync_copy(avg_hbm.at[pl.ds(base, bpw)], avg_v)
        cbu.wait()
        cbi.wait()

        lane = lax.iota(jnp.int32, 16)

        def fire(ucol16, icol16, lo, udst, idst):
            cps = []
            for k in range(lo, lo + 8):
                ucol = pl.multiple_of(ucol16[k], 128)
                icol = pl.multiple_of(icol16[k], 128)
                cps.append(pltpu.async_copy(
                    eu_hbm.at[:, pl.ds(ucol, 128)], udst.at[k - lo], sem_ue))
                cps.append(pltpu.async_copy(
                    ei_hbm.at[:, pl.ds(icol, 128)], idst.at[k - lo], sem_ie))
            return cps

        def accumulate(u16, i16, lo, usrc, isrc, acc):
            for k in range(lo, lo + 8):
                kk = jnp.full((16,), k - lo, jnp.int32)
                uc = jnp.broadcast_to(u16[k] & 127, (16,))
                ic = jnp.broadcast_to(i16[k] & 127, (16,))
                uf = plsc.load_gather(usrc, [kk, lane, uc])
                vf = plsc.load_gather(isrc, [kk, lane, ic])
                dot = jnp.sum(uf * vf, axis=0)
                acc = jnp.where(lane == k, acc + dot, acc)
            return acc

        def grp(g, carry):
            s = pl.ds(g * 16, 16)
            u16 = uidx_v[s]
            i16 = iidx_v[s]
            ucol16 = (u16 >> 7) * 128
            icol16 = (i16 >> 7) * 128
            cps_a = fire(ucol16, icol16, 0, ua_v, ia_v)
            cps_b = fire(ucol16, icol16, 8, ub_v, ib_v)
            acc = avg_v[s] + bu_v[s] + bi_v[s]
            for cp in cps_a:
                cp.wait()
            acc = accumulate(u16, i16, 0, ua_v, ia_v, acc)
            for cp in cps_b:
                cp.wait()
            acc = accumulate(u16, i16, 8, ub_v, ib_v, acc)
            out_v[s] = acc
            return carry

        lax.fori_loop(0, ngrp, grp, 0)

        pltpu.sync_copy(out_v, out_hbm.at[pl.ds(base, bpw)])

    return cdcf_kernel


def kernel(user, item, average, embed_user, embed_item, user_bias, item_bias):
    user = user.astype(jnp.int32)
    item = item.astype(jnp.int32)
    fn = _build(user.shape[0])
    return fn(user, item, average, embed_user.T, embed_item.T,
              user_bias, item_bias)


# speedup vs baseline: 18.6114x; 1.0273x over previous
import functools

import jax
import jax.numpy as jnp
from jax import lax
from jax.experimental import pallas as pl
from jax.experimental.pallas import tpu as pltpu
from jax.experimental.pallas import tpu_sc as plsc

_F = 16


@functools.lru_cache(maxsize=None)
def _build(batch):
    info = plsc.get_sparse_core_info()
    nc, ns = info.num_cores, info.num_subcores
    nw = nc * ns
    assert batch % (8 * nw) == 0
    bpw = batch // nw
    ngrp = bpw // 16

    mesh = plsc.VectorSubcoreMesh(core_axis_name="c", subcore_axis_name="s")

    @functools.partial(
        pl.kernel,
        mesh=mesh,
        out_type=jax.ShapeDtypeStruct((batch,), jnp.float32),
        compiler_params=pltpu.CompilerParams(needs_layout_passes=False),
        scratch_types=[
            pltpu.VMEM((bpw,), jnp.int32),
            pltpu.VMEM((bpw,), jnp.int32),
            pltpu.VMEM((16, 8, 128), jnp.float32),
            pltpu.VMEM((16, 8, 128), jnp.float32),
            pltpu.VMEM((16, 8, 128), jnp.float32),
            pltpu.VMEM((16, 8, 128), jnp.float32),
            pltpu.VMEM((bpw,), jnp.float32),
            pltpu.VMEM((bpw,), jnp.float32),
            pltpu.VMEM((bpw,), jnp.float32),
            pltpu.VMEM((bpw,), jnp.float32),
            pltpu.SemaphoreType.DMA,
            pltpu.SemaphoreType.DMA,
            pltpu.SemaphoreType.DMA,
        ],
    )
    def cdcf_kernel(user_hbm, item_hbm, avg_hbm, eu_hbm, ei_hbm,
                    bu_hbm, bi_hbm, out_hbm,
                    uidx_v, iidx_v, ux_v, ix_v, uy_v, iy_v,
                    bu_v, bi_v, avg_v, out_v,
                    sem_x, sem_y, sem_b):
        wid = lax.axis_index("s") * nc + lax.axis_index("c")
        base = wid * bpw

        pltpu.sync_copy(user_hbm.at[pl.ds(base, bpw)], uidx_v)
        pltpu.sync_copy(item_hbm.at[pl.ds(base, bpw)], iidx_v)

        cbu = pltpu.async_copy(bu_hbm.at[uidx_v], bu_v, sem_b)
        cbi = pltpu.async_copy(bi_hbm.at[iidx_v], bi_v, sem_b)

        pltpu.sync_copy(avg_hbm.at[pl.ds(base, bpw)], avg_v)
        cbu.wait()
        cbi.wait()

        lane = lax.iota(jnp.int32, 16)
        lane8 = lane & 7

        def cols(g):
            s = pl.ds(g * 16, 16)
            return uidx_v[s], iidx_v[s]

        def fire(g, h, udst, idst, sem):
            u16, i16 = cols(g)
            ucol16 = (u16 >> 7) * 128
            icol16 = (i16 >> 7) * 128
            for k in range(16):
                ucol = pl.multiple_of(ucol16[k], 128)
                icol = pl.multiple_of(icol16[k], 128)
                pltpu.async_copy(
                    eu_hbm.at[pl.ds(h * 8, 8), pl.ds(ucol, 128)],
                    udst.at[k], sem)
                pltpu.async_copy(
                    ei_hbm.at[pl.ds(h * 8, 8), pl.ds(icol, 128)],
                    idst.at[k], sem)

        def drain(udst, idst, sem):
            for k in range(16):
                pltpu.make_async_copy(
                    eu_hbm.at[pl.ds(0, 8), pl.ds(0, 128)],
                    udst.at[k], sem).wait()
                pltpu.make_async_copy(
                    ei_hbm.at[pl.ds(0, 8), pl.ds(0, 128)],
                    idst.at[k], sem).wait()

        def accumulate(g, usrc, isrc, acc):
            u16, i16 = cols(g)
            for k in range(16):
                kk = jnp.full((16,), k, jnp.int32)
                uc = jnp.broadcast_to(u16[k] & 127, (16,))
                ic = jnp.broadcast_to(i16[k] & 127, (16,))
                uf = plsc.load_gather(usrc, [kk, lane8, uc])
                vf = plsc.load_gather(isrc, [kk, lane8, ic])
                prod = jnp.where(lane < 8, uf * vf, 0.0)
                dot = jnp.sum(prod, axis=0)
                acc = jnp.where(lane == k, acc + dot, acc)
            return acc

        fire(0, 0, ux_v, ix_v, sem_x)

        def grp(g, carry):
            s = pl.ds(g * 16, 16)
            fire(g, 1, uy_v, iy_v, sem_y)
            acc = avg_v[s] + bu_v[s] + bi_v[s]
            drain(ux_v, ix_v, sem_x)
            acc = accumulate(g, ux_v, ix_v, acc)

            @pl.when(g + 1 < ngrp)
            def _():
                fire(g + 1, 0, ux_v, ix_v, sem_x)

            drain(uy_v, iy_v, sem_y)
            acc = accumulate(g, uy_v, iy_v, acc)
            out_v[s] = acc
            return carry

        lax.fori_loop(0, ngrp, grp, 0)

        pltpu.sync_copy(out_v, out_hbm.at[pl.ds(base, bpw)])

    return cdcf_kernel


def kernel(user, item, average, embed_user, embed_item, user_bias, item_bias):
    user = user.astype(jnp.int32)
    item = item.astype(jnp.int32)
    fn = _build(user.shape[0])
    return fn(user, item, average, embed_user.T, embed_item.T,
              user_bias, item_bias)
